# Initial kernel scaffold; baseline (speedup 1.0000x reference)
import jax
import jax.numpy as jnp
from jax.experimental import pallas as pl


def kernel(predicts, region_label, affinity_label):
    raise NotImplementedError("write your pallas kernel here")


# speedup vs baseline: 86.8762x; 86.8762x over previous
import jax
import jax.numpy as jnp
from jax.experimental import pallas as pl

_B, _H, _W = 32, 512, 512
_ROWS = _B * _H
_ROW_BLOCK = 512
_STEPS = _ROWS // _ROW_BLOCK
_N_PER_CHANNEL = _B * _H * _W
_SCALE = 1.0 / float(_N_PER_CHANNEL)


def _mse_sum_kernel(pred_ref, reg_ref, aff_ref, out_ref):
    pred = pred_ref[...]
    r = reg_ref[...]
    a = aff_ref[...]
    x = jnp.concatenate([r, a], axis=1)
    src = jax.lax.broadcasted_iota(jnp.int32, (2 * _W, 2 * _W), 0)
    dst = jax.lax.broadcasted_iota(jnp.int32, (2 * _W, 2 * _W), 1)
    p_mat = ((dst == 2 * src) | (dst == 2 * src - (2 * _W - 1))).astype(jnp.float32)
    lbl = jax.lax.dot_general(x, p_mat, (((1,), (0,)), ((), ())),
                              preferred_element_type=jnp.float32)
    d = pred - lbl
    s = jnp.sum(d * d) * _SCALE

    @pl.when(pl.program_id(0) == 0)
    def _init():
        out_ref[...] = jnp.zeros_like(out_ref)

    out_ref[...] = out_ref[...] + s


def kernel(predicts, region_label, affinity_label):
    pred2d = predicts.reshape(_ROWS, 2 * _W)
    reg2d = region_label.reshape(_ROWS, _W)
    aff2d = affinity_label.reshape(_ROWS, _W)
    out = pl.pallas_call(
        _mse_sum_kernel,
        grid=(_STEPS,),
        in_specs=[
            pl.BlockSpec((_ROW_BLOCK, 2 * _W), lambda i: (i, 0)),
            pl.BlockSpec((_ROW_BLOCK, _W), lambda i: (i, 0)),
            pl.BlockSpec((_ROW_BLOCK, _W), lambda i: (i, 0)),
        ],
        out_specs=pl.BlockSpec((1, 1), lambda i: (0, 0)),
        out_shape=jax.ShapeDtypeStruct((1, 1), jnp.float32),
    )(pred2d, reg2d, aff2d)
    return out[0, 0]
